# Initial kernel scaffold; baseline (speedup 1.0000x reference)
import jax
import jax.numpy as jnp
from jax.experimental import pallas as pl


def kernel(x, emb_weight):
    raise NotImplementedError("write your pallas kernel here")


# speedup vs baseline: 3.0338x; 3.0338x over previous
import jax
import jax.numpy as jnp
from jax.experimental import pallas as pl


def kernel(x, emb_weight):
    seq_len = x.shape[1]
    d = emb_weight.shape[1]
    blk = 512

    def body(in_ref, out_ref):
        out_ref[...] = in_ref[...]

    return pl.pallas_call(
        body,
        grid=(seq_len // blk,),
        in_specs=[pl.BlockSpec((blk, d), lambda i: (i, 0))],
        out_specs=pl.BlockSpec((blk, d), lambda i: (i, 0)),
        out_shape=jax.ShapeDtypeStruct((seq_len, d), emb_weight.dtype),
    )(emb_weight)
